# baseline (speedup 1.0000x reference)
import functools

import jax
import jax.numpy as jnp
from jax import lax
from jax.experimental import pallas as pl
from jax.experimental.pallas import tpu as pltpu
from jax.experimental.pallas import tpu_sc as plsc

BATCH = 16384
EMBED = 32
NUM_ROWS = 1000000

_NC = 2
_NS = 16
_NW = _NC * _NS
_B_PER_W = BATCH // _NW

_mesh = plsc.VectorSubcoreMesh(core_axis_name="c", subcore_axis_name="s")


@functools.partial(
    pl.kernel,
    mesh=_mesh,
    out_type=jax.ShapeDtypeStruct((BATCH, EMBED), jnp.float32),
    scratch_types=[
        pltpu.VMEM((_B_PER_W,), jnp.int32),
        pltpu.VMEM((_B_PER_W, EMBED), jnp.float32),
        pltpu.SemaphoreType.DMA,
    ],
    compiler_params=pltpu.CompilerParams(use_tc_tiling_on_sc=False),
)
def _gather_kernel(idx_hbm, table_hbm, out_hbm, idx_v, rows_v, sem):
    wid = lax.axis_index("s") * _NC + lax.axis_index("c")
    base = wid * _B_PER_W
    pltpu.sync_copy(idx_hbm.at[pl.ds(base, _B_PER_W)], idx_v)
    pltpu.async_copy(table_hbm.at[idx_v], rows_v, sem).wait()
    pltpu.sync_copy(rows_v, out_hbm.at[pl.ds(base, _B_PER_W)])


def kernel(landmark_i, table):
    return _gather_kernel(landmark_i.astype(jnp.int32), table)


# speedup vs baseline: 2.1369x; 2.1369x over previous
import functools

import jax
import jax.numpy as jnp
from jax import lax
from jax.experimental import pallas as pl
from jax.experimental.pallas import tpu as pltpu
from jax.experimental.pallas import tpu_sc as plsc

BATCH = 16384
EMBED = 32
ROWS = 1000000

_NC = 2
_NS = 16
_NW = _NC * _NS

_CH = 512
_MAIN = (ROWS // _CH) * _CH
_NCHUNK = _MAIN // _CH
_TAIL = ROWS - _MAIN
_OUT_PAD = 32
_HCAP = BATCH + 16

_mesh = plsc.VectorSubcoreMesh(core_axis_name="c", subcore_axis_name="s")


@functools.partial(
    pl.kernel,
    mesh=_mesh,
    compiler_params=pltpu.CompilerParams(needs_layout_passes=False),
    out_type=jax.ShapeDtypeStruct((BATCH + _OUT_PAD, 128), jnp.float32),
    scratch_types=[
        pltpu.VMEM((BATCH,), jnp.int32),
        pltpu.VMEM((_HCAP,), jnp.int32),
        pltpu.VMEM((_HCAP,), jnp.int32),
        pltpu.VMEM((_HCAP,), jnp.int32),
        pltpu.VMEM((_HCAP,), jnp.int32),
        pltpu.VMEM((EMBED, _CH), jnp.float32),
        pltpu.VMEM((_TAIL, EMBED), jnp.float32),
        pltpu.VMEM((16, 128), jnp.float32),
        pltpu.SemaphoreType.DMA,
        pltpu.SemaphoreType.DMA,
    ],
)
def _scan_kernel(idx_hbm, tt_hbm, aux_hbm, out_hbm, idx_v, hpos_v, hcol_v,
                 cpos_v, ccol_v, chunk_v, aux_v, rowbuf_v, csem, ssem):
    wid = lax.axis_index("s") * _NC + lax.axis_index("c")
    pltpu.sync_copy(idx_hbm, idx_v)
    pltpu.sync_copy(aux_hbm, aux_v)
    lanes = lax.iota(jnp.int32, 16)

    def prescan(g, offs):
        off_main, off_aux = offs
        v = idx_v[pl.ds(g * 16, 16)]
        posv = g * 16 + lanes
        in_main = ((v // _CH) % _NW) == wid
        m1 = jnp.logical_and(in_main, v < _MAIN)
        m2 = jnp.logical_and(v >= _MAIN, (g // (BATCH // (16 * _NW))) == wid)
        c1 = plsc.cumsum(jnp.where(m1, 1, 0).astype(jnp.int32))
        d1 = off_main + c1 - 1
        plsc.store_scatter(hpos_v, [d1], posv, mask=m1)
        plsc.store_scatter(hcol_v, [d1], v, mask=m1)
        n1 = lax.reduce_max(c1, (0,))
        c2 = plsc.cumsum(jnp.where(m2, 1, 0).astype(jnp.int32))
        d2 = off_aux + c2 - 1
        plsc.store_scatter(cpos_v, [d2], posv, mask=m2)
        plsc.store_scatter(ccol_v, [d2], v - _MAIN, mask=m2)
        n2 = lax.reduce_max(c2, (0,))
        return off_main + n1, off_aux + n2

    nhit, naux = lax.fori_loop(0, BATCH // 16, prescan, (0, 0))

    def aux_group(e, carry):
        pv = cpos_v[pl.ds(e * 16, 16)]
        jv = ccol_v[pl.ds(e * 16, 16)]
        valid = (e * 16 + lanes) < naux
        pvs = jnp.where(valid, pv, BATCH + lanes)
        jvs = jnp.where(valid, jv, 0)
        for s in range(EMBED):
            val = plsc.load_gather(aux_v, [jvs, jnp.full((16,), s, jnp.int32)])
            plsc.store_scatter(rowbuf_v, [lanes, jnp.full((16,), s, jnp.int32)], val)
        pltpu.async_copy(rowbuf_v, out_hbm.at[pvs], ssem).wait()
        return carry

    lax.fori_loop(0, (naux + 15) // 16, aux_group, 0)

    n_my_chunks = (_NCHUNK - wid + _NW - 1) // _NW

    def chunk_body(k, carry):
        cid = wid + k * _NW
        c0 = pl.multiple_of(cid * _CH, _CH)
        cp = pltpu.async_copy(tt_hbm.at[:, pl.ds(c0, _CH)], chunk_v, csem)

        def compact(h, off2):
            pv = hpos_v[pl.ds(h * 16, 16)]
            jv = hcol_v[pl.ds(h * 16, 16)]
            valid = (h * 16 + lanes) < nhit
            m = jnp.logical_and(valid, (jv // _CH) == cid)
            cc = plsc.cumsum(jnp.where(m, 1, 0).astype(jnp.int32))
            d = off2 + cc - 1
            plsc.store_scatter(cpos_v, [d], pv, mask=m)
            plsc.store_scatter(ccol_v, [d], jv - c0, mask=m)
            return off2 + lax.reduce_max(cc, (0,))

        n2 = lax.fori_loop(0, (nhit + 15) // 16, compact, 0)
        cp.wait()

        def extract(e, carry2):
            pv = cpos_v[pl.ds(e * 16, 16)]
            jv = ccol_v[pl.ds(e * 16, 16)]
            valid = (e * 16 + lanes) < n2
            pvs = jnp.where(valid, pv, BATCH + lanes)
            jvs = jnp.where(valid, jv, 0)
            for s in range(EMBED):
                val = plsc.load_gather(chunk_v, [jnp.full((16,), s, jnp.int32), jvs])
                plsc.store_scatter(rowbuf_v, [lanes, jnp.full((16,), s, jnp.int32)], val)
            pltpu.async_copy(rowbuf_v, out_hbm.at[pvs], ssem).wait()
            return carry2

        lax.fori_loop(0, (n2 + 15) // 16, extract, 0)
        return carry

    lax.fori_loop(0, n_my_chunks, chunk_body, 0)


def kernel(landmark_i, table):
    tt = table.T
    aux = table[_MAIN:]
    res = _scan_kernel(landmark_i.astype(jnp.int32), tt, aux)
    return res[:BATCH, :EMBED]
